# baseline (speedup 1.0000x reference)
import functools

import jax
import jax.numpy as jnp
from jax import lax
from jax.experimental import pallas as pl
from jax.experimental.pallas import tpu as pltpu
from jax.experimental.pallas import tpu_sc as plsc

N = 10000
D = 128
G = 64
NC = 2
NS = 16
NW = NC * NS
CHUNK = 128
NP = 10240
NPH = NP // NC
NPH2 = 5248
PT = NP // NS
PA = NPH2 // NS
PQ = NP // NW
TRASH = N

_mesh = plsc.VectorSubcoreMesh(core_axis_name="c", subcore_axis_name="s")


def _deg_kernel(epc):
    @functools.partial(
        pl.kernel,
        mesh=_mesh,
        out_type=jax.ShapeDtypeStruct((NC, NP * 16), jnp.float32),
        scratch_types=[
            pltpu.VMEM((epc, CHUNK), jnp.int32),
            pltpu.VMEM((CHUNK,), jnp.float32),
            pltpu.VMEM_SHARED((NP * 16,), jnp.float32),
        ],
    )
    def k(dst16_h, zeros_h, ones_h, out_h, dstv, onesv, acc):
        c = lax.axis_index("c")
        s = lax.axis_index("s")
        wid = c * NS + s
        pltpu.sync_copy(zeros_h, acc.at[pl.ds(s * PT * 16, PT * 16)])
        pltpu.sync_copy(ones_h, onesv)
        pltpu.sync_copy(dst16_h.at[wid], dstv)
        plsc.subcore_barrier()
        for j in range(epc):
            pltpu.sync_copy(onesv, acc.at[dstv.at[j]], add=True)
        plsc.subcore_barrier()
        pltpu.sync_copy(acc.at[pl.ds(s * PT * 16, PT * 16)],
                        out_h.at[c, pl.ds(s * PT * 16, PT * 16)])

    return k


def _agg_kernel(epc2, f):
    H = D // 2

    @functools.partial(
        pl.kernel,
        mesh=_mesh,
        out_type=jax.ShapeDtypeStruct((NC, NPH2 * H), jnp.float32),
        scratch_types=[
            pltpu.VMEM((epc2, CHUNK), jnp.int32),
            pltpu.VMEM((epc2, CHUNK), jnp.int32),
            pltpu.VMEM((CHUNK, D), jnp.float32),
            pltpu.VMEM((CHUNK, D), jnp.float32),
            pltpu.VMEM((CHUNK, D), jnp.int32),
            pltpu.VMEM((CHUNK, D), jnp.int32),
            pltpu.VMEM_SHARED((NPH2 * H,), jnp.float32),
            pltpu.SemaphoreType.DMA,
            pltpu.SemaphoreType.DMA,
            pltpu.SemaphoreType.DMA,
            pltpu.SemaphoreType.DMA,
            pltpu.SemaphoreType.DMA,
        ],
    )
    def k(g_h, pk_h, idxtab_h, zeros_h, out_h,
          srcv, dstv, rb0, rb1, ib0, ib1, acc, semg0, semg1, semi0, semi1, sems):
        c = lax.axis_index("c")
        s = lax.axis_index("s")
        pltpu.sync_copy(pk_h.at[s], dstv)
        pltpu.sync_copy(zeros_h, acc.at[pl.ds(s * PA * H, PA * H)])
        base = c * NPH
        trash = NPH + s

        def obody(j, carry):
            for q in range(CHUNK // 16):
                sl = pl.ds(q * 16, 16)
                pk = dstv[j, sl]
                srcv[j, sl] = pk & 16383
                dl = lax.shift_right_logical(pk, 14) - base
                ok = (dl >= 0) & (dl < NPH)
                dstv[j, sl] = jnp.where(ok, dl, trash)
            return carry

        lax.fori_loop(0, epc2, obody, 0)
        plsc.subcore_barrier()
        rbufs = (rb0, rb1)
        ibufs = (ib0, ib1)
        semg = (semg0, semg1)
        semi = (semi0, semi1)
        pltpu.async_copy(g_h.at[srcv.at[0]], rb0, semg0)
        pltpu.async_copy(idxtab_h.at[dstv.at[0]], ib0, semi0)
        for j in range(epc2):
            b = j % 2
            pltpu.make_async_copy(g_h.at[srcv.at[j]], rbufs[b], semg[b]).wait()
            pltpu.make_async_copy(idxtab_h.at[dstv.at[j]], ibufs[b], semi[b]).wait()
            if j + 1 < epc2:
                nb = (j + 1) % 2
                pltpu.async_copy(g_h.at[srcv.at[j + 1]], rbufs[nb], semg[nb])
                pltpu.async_copy(idxtab_h.at[dstv.at[j + 1]], ibufs[nb], semi[nb])
            rb, ib = rbufs[b], ibufs[b]
            hs = pl.ds(f * H, H)

            def fire(e2, carry):
                pltpu.async_copy(rb.at[e2, hs], acc.at[ib.at[e2, hs]],
                                 sems, add=True)
                return carry

            lax.fori_loop(0, CHUNK, fire, 0)
            pltpu.make_async_copy(
                g_h.at[pl.ds(0, CHUNK // 2)], rb.at[pl.ds(0, CHUNK // 2)],
                sems).wait()
        plsc.subcore_barrier()
        pltpu.sync_copy(acc.at[pl.ds(s * PA * H, PA * H)],
                        out_h.at[c, pl.ds(s * PA * H, PA * H)])

    return k


@functools.partial(
    pl.kernel,
    mesh=_mesh,
    out_type=(
        jax.ShapeDtypeStruct((NW, G, D), jnp.float32),
        jax.ShapeDtypeStruct((NW, G, D), jnp.float32),
        jax.ShapeDtypeStruct((NW, G, 16), jnp.float32),
    ),
    scratch_types=[
        pltpu.VMEM((PQ, D), jnp.float32),
        pltpu.VMEM((PQ + 16,), jnp.int32),
        pltpu.VMEM((G, D), jnp.float32),
        pltpu.VMEM((G, D), jnp.float32),
        pltpu.VMEM((G, 16), jnp.float32),
    ],
)
def _pool_kernel(h_h, batch_h, zeros_h, zeros16_h, neginf_h, out_s, out_m, out_c,
                 rowsv, batchv, asum, amax, acnt):
    c = lax.axis_index("c")
    s = lax.axis_index("s")
    wid = c * NS + s
    base = wid * PQ
    pltpu.sync_copy(h_h.at[pl.ds(base, PQ)], rowsv)
    pltpu.sync_copy(batch_h.at[pl.ds(base, PQ)], batchv.at[pl.ds(0, PQ)])
    pltpu.sync_copy(zeros_h.at[pl.ds(0, G)], asum)
    pltpu.sync_copy(zeros16_h.at[pl.ds(0, G)], acnt)
    pltpu.sync_copy(neginf_h, amax)
    cnt = jnp.minimum(PQ, jnp.maximum(N - base, 0))
    ones16 = jnp.ones((16,), jnp.float32)

    def body(r, carry):
        seg = batchv[pl.ds(r, 16)][0]
        for cc in range(D // 16):
            sl = pl.ds(cc * 16, 16)
            v = rowsv[r, sl]
            asum[seg, sl] = asum[seg, sl] + v
            amax[seg, sl] = jnp.maximum(amax[seg, sl], v)
        acnt[seg, :] = acnt[seg, :] + ones16
        return carry

    lax.fori_loop(0, cnt, body, 0)
    pltpu.sync_copy(asum, out_s.at[wid])
    pltpu.sync_copy(amax, out_m.at[wid])
    pltpu.sync_copy(acnt, out_c.at[wid])


def _prep_call(x_p, degp, W1):
    R = PT

    def body(x_ref, d0_ref, d1_ref, w_ref, g_ref, dinv_ref):
        deg = 1.0 + d0_ref[0][:, 0:1] + d1_ref[0][:, 0:1]
        dinv = lax.rsqrt(jnp.maximum(deg, 1.0))
        h = jnp.dot(x_ref[...], w_ref[...], preferred_element_type=jnp.float32)
        g_ref[...] = dinv * h
        dinv_ref[...] = dinv

    return pl.pallas_call(
        body,
        grid=(NP // R,),
        in_specs=[
            pl.BlockSpec((R, D), lambda i: (i, 0)),
            pl.BlockSpec((1, R, 16), lambda i: (0, i, 0)),
            pl.BlockSpec((1, R, 16), lambda i: (1, i, 0)),
            pl.BlockSpec((D, D), lambda i: (0, 0)),
        ],
        out_specs=[
            pl.BlockSpec((R, D), lambda i: (i, 0)),
            pl.BlockSpec((R, 1), lambda i: (i, 0)),
        ],
        out_shape=[
            jax.ShapeDtypeStruct((NP, D), jnp.float32),
            jax.ShapeDtypeStruct((NP, 1), jnp.float32),
        ],
    )(x_p, degp, degp, W1)


def _layer_call(p0, p1, g, dinv, b, W_next):
    R = PT
    H = D // 2
    nbh = NPH // R
    has_w = W_next is not None

    def body(p0_ref, p1_ref, g_ref, dinv_ref, b_ref, *rest):
        pf = jnp.concatenate([p0_ref[0], p1_ref[0]], axis=1)
        t = jnp.tanh(dinv_ref[...] * (pf + g_ref[...]) + b_ref[...])
        if has_w:
            w_ref, out_ref = rest
            out_ref[...] = dinv_ref[...] * jnp.dot(
                t, w_ref[...], preferred_element_type=jnp.float32)
        else:
            (out_ref,) = rest
            out_ref[...] = t

    in_specs = [
        pl.BlockSpec((1, R, H), lambda i: (i // nbh, i % nbh, 0)),
        pl.BlockSpec((1, R, H), lambda i: (i // nbh, i % nbh, 0)),
        pl.BlockSpec((R, D), lambda i: (i, 0)),
        pl.BlockSpec((R, 1), lambda i: (i, 0)),
        pl.BlockSpec((1, D), lambda i: (0, 0)),
    ]
    args = [p0, p1, g, dinv, b.reshape(1, D)]
    if has_w:
        in_specs.append(pl.BlockSpec((D, D), lambda i: (0, 0)))
        args.append(W_next)
    return pl.pallas_call(
        body,
        grid=(NP // R,),
        in_specs=in_specs,
        out_specs=pl.BlockSpec((R, D), lambda i: (i, 0)),
        out_shape=jax.ShapeDtypeStruct((NP, D), jnp.float32),
    )(*args)


def _readout_call(sums, maxs, cnts, R1_W, R1_b, R2_W, R2_b):
    def body(s_ref, m_ref, c_ref, w1_ref, b1_ref, w2_ref, b2_ref, out_ref):
        sp = s_ref[0]
        mp = m_ref[0]
        ct = c_ref[0]
        for i in range(1, NW):
            sp = sp + s_ref[i]
            mp = jnp.maximum(mp, m_ref[i])
            ct = ct + c_ref[i]
        cnt = ct[:, 0:1]
        mean = sp / jnp.maximum(cnt, 1.0)
        z = jnp.concatenate([sp, mp, mean], axis=1)
        z = jnp.dot(z, w1_ref[...], preferred_element_type=jnp.float32) + b1_ref[...]
        z = jnp.where(z >= 0, z, 0.01 * z)
        z = jnp.dot(z, w2_ref[...], preferred_element_type=jnp.float32) + b2_ref[...]
        out_ref[...] = jnp.where(z >= 0, z, 0.01 * z)

    h1 = R1_W.shape[1]
    out = R2_W.shape[1]
    return pl.pallas_call(
        body,
        out_shape=jax.ShapeDtypeStruct((G, out), jnp.float32),
    )(sums, maxs, cnts, R1_W, R1_b.reshape(1, h1), R2_W, R2_b.reshape(1, out))


def kernel(x, edge_index, batch, W1, b1, W2, b2, W3, b3, R1_W, R1_b, R2_W, R2_b):
    e = edge_index.shape[1]
    epc = -(-e // (NW * CHUNK))
    e_pad = NW * CHUNK * epc
    epc2 = NC * epc

    src = edge_index[0].astype(jnp.int32)
    dst = edge_index[1].astype(jnp.int32)
    src_p = jnp.concatenate([src, jnp.zeros((e_pad - e,), jnp.int32)])
    dst_p = jnp.concatenate([dst, jnp.full((e_pad - e,), TRASH, jnp.int32)])
    dst16 = (dst_p * 16).reshape(NW, epc, CHUNK)
    pk2 = (dst_p * 16384 + src_p).reshape(NS, epc2, CHUNK)
    x_p = jnp.pad(x, ((0, NP - N), (0, 0)))
    batch_p = jnp.pad(batch.astype(jnp.int32), (0, NP - N))
    H = D // 2
    v = jnp.arange(NPH2, dtype=jnp.int32)[:, None]
    io = jnp.arange(H, dtype=jnp.int32)[None, :]
    real = v * H + io
    junk = (NPH + (v % 128)) * H + io
    idxtab0 = jnp.concatenate([real, junk], axis=1)
    idxtab1 = jnp.concatenate([junk, real], axis=1)

    zeros_agg = jnp.zeros((PA * (D // 2),), jnp.float32)
    zeros_deg = jnp.zeros((PT * 16,), jnp.float32)
    zeros128 = jnp.zeros((G, D), jnp.float32)
    zeros16 = jnp.zeros((G, 16), jnp.float32)
    ones_deg = jnp.ones((CHUNK,), jnp.float32)
    neginf = jnp.full((G, D), -jnp.inf, jnp.float32)

    degp = _deg_kernel(epc)(dst16, zeros_deg, ones_deg).reshape(NC, NP, 16)
    g, dinv = _prep_call(x_p, degp, W1)

    agg0 = _agg_kernel(epc2, 0)
    agg1 = _agg_kernel(epc2, 1)

    def agg_call(gv):
        q0 = agg0(gv, pk2, idxtab0, zeros_agg).reshape(NC, NPH2, D // 2)
        q1 = agg1(gv, pk2, idxtab1, zeros_agg).reshape(NC, NPH2, D // 2)
        return q0, q1

    p0, p1 = agg_call(g)
    g = _layer_call(p0, p1, g, dinv, b1, W2)
    p0, p1 = agg_call(g)
    g = _layer_call(p0, p1, g, dinv, b2, W3)
    p0, p1 = agg_call(g)
    h3 = _layer_call(p0, p1, g, dinv, b3, None)

    sums, maxs, cnts = _pool_kernel(h3, batch_p, zeros128, zeros16, neginf)
    return _readout_call(sums, maxs, cnts, R1_W, R1_b, R2_W, R2_b)
